# baseline (speedup 1.0000x reference)
import jax
import jax.numpy as jnp
from jax import lax
from jax.experimental import pallas as pl

NT = 8192
NKV = NT // 2
NF_IN = 128
NF = 256
EPS = 1e-4

BQ = 512
NBQ = NT // BQ
BKV = NKV // NBQ


def _stats_to_affine(st, n, g, b):
    mu = st[0:1, :] / n
    var = st[1:2, :] / n - mu * mu
    a = g * lax.rsqrt(var + EPS)
    c = b - mu * a
    return a, c


def _acc_stats(st_ref, yb, i):
    @pl.when(i == 0)
    def _():
        st_ref[...] = jnp.zeros_like(st_ref)
    st_ref[0:1, :] += jnp.sum(yb, axis=0, keepdims=True)
    st_ref[1:2, :] += jnp.sum(yb * yb, axis=0, keepdims=True)


def _dot(a, b):
    return jnp.dot(a, b, preferred_element_type=jnp.float32)


def _s1(fb, feb, wp1, wkv, wq, x_o, q_o, kvp_o, st_o):
    i = pl.program_id(0)
    xb = _dot(fb[...], wp1[...])
    x_o[...] = xb
    q_o[...] = _dot(xb, wq[...])
    kb = _dot(_dot(feb[...], wp1[...]), wkv[...])
    kvp_o[...] = kb
    _acc_stats(st_o, kb, i)


def _s2(kvpb, st, g, b, wk, wv, k_o, v_o):
    a, c = _stats_to_affine(st[...], NKV, g[...], b[...])
    kn = kvpb[...] * a + c
    k_o[...] = _dot(kn, wk[...])
    v_o[...] = _dot(kn, wv[...])


def _s3(qb, k, v, wtrans, y_o, st_o):
    i = pl.program_id(0)
    s = lax.dot_general(qb[...], k[...], (((1,), (1,)), ((), ())),
                        preferred_element_type=jnp.float32)
    m = jnp.max(s, axis=1, keepdims=True)
    p = jnp.exp(s - m)
    d = jnp.sum(p, axis=1, keepdims=True)
    xr = _dot(p, v[...]) / d
    yb = _dot(xr, wtrans[...])
    y_o[...] = yb
    _acc_stats(st_o, yb, i)


def _s4(xb, yb, st, g, b, x2_o, st_o):
    i = pl.program_id(0)
    a, c = _stats_to_affine(st[...], NT, g[...], b[...])
    x2 = xb[...] + yb[...] * a + c
    x2_o[...] = x2
    _acc_stats(st_o, x2, i)


def _s5(x2b, st, g, b, wr1, h1_o, st_o):
    i = pl.program_id(0)
    a, c = _stats_to_affine(st[...], NT, g[...], b[...])
    h1 = _dot(jnp.maximum(x2b[...] * a + c, 0.0), wr1[...])
    h1_o[...] = h1
    _acc_stats(st_o, h1, i)


def _s6(h1b, x2b, st, g, b, wr2, x3_o, st_o):
    i = pl.program_id(0)
    a, c = _stats_to_affine(st[...], NT, g[...], b[...])
    x3 = x2b[...] + _dot(jnp.maximum(h1b[...] * a + c, 0.0), wr2[...])
    x3_o[...] = x3
    _acc_stats(st_o, x3, i)


def _s7(x3b, st, g, b, out_o):
    a, c = _stats_to_affine(st[...], NT, g[...], b[...])
    out_o[...] = jnp.maximum(x3b[...] * a + c, 0.0)


def _rows(bs):
    return pl.BlockSpec((bs, NF), lambda i: (i, 0))


def _full(shape):
    return pl.BlockSpec(shape, lambda i: (0, 0))


_ST = jax.ShapeDtypeStruct((8, NF), jnp.float32)
_STSPEC = pl.BlockSpec((8, NF), lambda i: (0, 0))


def kernel(flat, cu_seqlens, W_p1, W_kv, g_kv, b_kv, W_q, W_k, W_v,
           W_trans, g_an, b_an, g1, b1, W_r1, g2, b2, W_r2, g3, b3):
    del cu_seqlens
    f32 = jnp.float32
    flat_even = flat[::2]
    g_kv2, b_kv2 = g_kv.reshape(1, NF), b_kv.reshape(1, NF)
    g_an2, b_an2 = g_an.reshape(1, NF), b_an.reshape(1, NF)
    g12, b12 = g1.reshape(1, NF), b1.reshape(1, NF)
    g22, b22 = g2.reshape(1, NF), b2.reshape(1, NF)
    g32, b32 = g3.reshape(1, NF), b3.reshape(1, NF)

    x, q, kvp, st_kv = pl.pallas_call(
        _s1,
        grid=(NBQ,),
        in_specs=[pl.BlockSpec((BQ, NF_IN), lambda i: (i, 0)),
                  pl.BlockSpec((BKV, NF_IN), lambda i: (i, 0)),
                  _full((NF_IN, NF)), _full((NF, NF)), _full((NF, NF))],
        out_specs=[_rows(BQ), _rows(BQ), _rows(BKV), _STSPEC],
        out_shape=[jax.ShapeDtypeStruct((NT, NF), f32),
                   jax.ShapeDtypeStruct((NT, NF), f32),
                   jax.ShapeDtypeStruct((NKV, NF), f32), _ST],
    )(flat, flat_even, W_p1, W_kv, W_q)

    k, v = pl.pallas_call(
        _s2,
        grid=(NKV // BQ,),
        in_specs=[_rows(BQ), _STSPEC, _full((1, NF)), _full((1, NF)),
                  _full((NF, NF)), _full((NF, NF))],
        out_specs=[_rows(BQ), _rows(BQ)],
        out_shape=[jax.ShapeDtypeStruct((NKV, NF), f32),
                   jax.ShapeDtypeStruct((NKV, NF), f32)],
    )(kvp, st_kv, g_kv2, b_kv2, W_k, W_v)

    y, st_y = pl.pallas_call(
        _s3,
        grid=(NBQ,),
        in_specs=[_rows(BQ), _full((NKV, NF)), _full((NKV, NF)),
                  _full((NF, NF))],
        out_specs=[_rows(BQ), _STSPEC],
        out_shape=[jax.ShapeDtypeStruct((NT, NF), f32), _ST],
    )(q, k, v, W_trans)

    x2, st_x2 = pl.pallas_call(
        _s4,
        grid=(NBQ,),
        in_specs=[_rows(BQ), _rows(BQ), _STSPEC, _full((1, NF)),
                  _full((1, NF))],
        out_specs=[_rows(BQ), _STSPEC],
        out_shape=[jax.ShapeDtypeStruct((NT, NF), f32), _ST],
    )(x, y, st_y, g_an2, b_an2)

    h1, st_h1 = pl.pallas_call(
        _s5,
        grid=(NBQ,),
        in_specs=[_rows(BQ), _STSPEC, _full((1, NF)), _full((1, NF)),
                  _full((NF, NF))],
        out_specs=[_rows(BQ), _STSPEC],
        out_shape=[jax.ShapeDtypeStruct((NT, NF), f32), _ST],
    )(x2, st_x2, g12, b12, W_r1)

    x3, st_x3 = pl.pallas_call(
        _s6,
        grid=(NBQ,),
        in_specs=[_rows(BQ), _rows(BQ), _STSPEC, _full((1, NF)),
                  _full((1, NF)), _full((NF, NF))],
        out_specs=[_rows(BQ), _STSPEC],
        out_shape=[jax.ShapeDtypeStruct((NT, NF), f32), _ST],
    )(h1, x2, st_h1, g22, b22, W_r2)

    out = pl.pallas_call(
        _s7,
        grid=(NBQ,),
        in_specs=[_rows(BQ), _STSPEC, _full((1, NF)), _full((1, NF))],
        out_specs=_rows(BQ),
        out_shape=jax.ShapeDtypeStruct((NT, NF), f32),
    )(x3, st_x3, g32, b32)

    return out


# speedup vs baseline: 1.0132x; 1.0132x over previous
import jax
import jax.numpy as jnp
from jax import lax
from jax.experimental import pallas as pl

NT = 8192
NKV = NT // 2
NF_IN = 128
NF = 256
EPS = 1e-4

BQ = 512
NBQ = NT // BQ
BKV = NKV // NBQ


def _stats_to_affine(st, n, g, b):
    mu = st[0:1, :] / n
    var = st[1:2, :] / n - mu * mu
    a = g * lax.rsqrt(var + EPS)
    c = b - mu * a
    return a, c


def _acc_stats(st_ref, yb, i):
    @pl.when(i == 0)
    def _():
        st_ref[...] = jnp.zeros_like(st_ref)
    st_ref[0:1, :] += jnp.sum(yb, axis=0, keepdims=True)
    st_ref[1:2, :] += jnp.sum(yb * yb, axis=0, keepdims=True)


def _bf(a):
    return a.astype(jnp.bfloat16)


def _dot(a, b):
    return jnp.dot(_bf(a), _bf(b), preferred_element_type=jnp.float32)


_bdot = _dot


def _s1(fb, feb, wp1, wkv, wq, x_o, q_o, kvp_o, st_o):
    i = pl.program_id(0)
    xb = _dot(fb[...], wp1[...])
    x_o[...] = xb
    q_o[...] = _bf(_dot(xb, wq[...]))
    kb = _dot(_dot(feb[...], wp1[...]), wkv[...])
    kvp_o[...] = kb
    _acc_stats(st_o, kb, i)


def _s2(kvpb, st, g, b, wk, wv, k_o, v_o):
    a, c = _stats_to_affine(st[...], NKV, g[...], b[...])
    kn = kvpb[...] * a + c
    k_o[...] = _bf(_dot(kn, wk[...]))
    v_o[...] = _bf(_dot(kn, wv[...]))


def _s3(qb, k, v, wtrans, y_o, st_o):
    i = pl.program_id(0)
    s = lax.dot_general(qb[...], k[...], (((1,), (1,)), ((), ())),
                        preferred_element_type=jnp.float32)
    m = jnp.max(s, axis=1, keepdims=True)
    p = jnp.exp(s - m)
    d = jnp.sum(p, axis=1, keepdims=True)
    xr = jnp.dot(_bf(p), v[...], preferred_element_type=jnp.float32) / d
    yb = _dot(xr, wtrans[...])
    y_o[...] = yb
    _acc_stats(st_o, yb, i)


def _s4(xb, yb, st, g, b, x2_o, st_o):
    i = pl.program_id(0)
    a, c = _stats_to_affine(st[...], NT, g[...], b[...])
    x2 = xb[...] + yb[...] * a + c
    x2_o[...] = x2
    _acc_stats(st_o, x2, i)


def _s5(x2b, st, g, b, wr1, h1_o, st_o):
    i = pl.program_id(0)
    a, c = _stats_to_affine(st[...], NT, g[...], b[...])
    h1 = _dot(jnp.maximum(x2b[...] * a + c, 0.0), wr1[...])
    h1_o[...] = h1
    _acc_stats(st_o, h1, i)


def _s6(h1b, x2b, st, g, b, wr2, x3_o, st_o):
    i = pl.program_id(0)
    a, c = _stats_to_affine(st[...], NT, g[...], b[...])
    x3 = x2b[...] + _dot(jnp.maximum(h1b[...] * a + c, 0.0), wr2[...])
    x3_o[...] = x3
    _acc_stats(st_o, x3, i)


def _s7(x3b, st, g, b, out_o):
    a, c = _stats_to_affine(st[...], NT, g[...], b[...])
    out_o[...] = jnp.maximum(x3b[...] * a + c, 0.0)


def _rows(bs):
    return pl.BlockSpec((bs, NF), lambda i: (i, 0))


def _full(shape):
    return pl.BlockSpec(shape, lambda i: (0, 0))


_ST = jax.ShapeDtypeStruct((8, NF), jnp.float32)
_STSPEC = pl.BlockSpec((8, NF), lambda i: (0, 0))


def kernel(flat, cu_seqlens, W_p1, W_kv, g_kv, b_kv, W_q, W_k, W_v,
           W_trans, g_an, b_an, g1, b1, W_r1, g2, b2, W_r2, g3, b3):
    del cu_seqlens
    f32 = jnp.float32
    flat_even = flat[::2]
    g_kv2, b_kv2 = g_kv.reshape(1, NF), b_kv.reshape(1, NF)
    g_an2, b_an2 = g_an.reshape(1, NF), b_an.reshape(1, NF)
    g12, b12 = g1.reshape(1, NF), b1.reshape(1, NF)
    g22, b22 = g2.reshape(1, NF), b2.reshape(1, NF)
    g32, b32 = g3.reshape(1, NF), b3.reshape(1, NF)

    x, q, kvp, st_kv = pl.pallas_call(
        _s1,
        grid=(NBQ,),
        in_specs=[pl.BlockSpec((BQ, NF_IN), lambda i: (i, 0)),
                  pl.BlockSpec((BKV, NF_IN), lambda i: (i, 0)),
                  _full((NF_IN, NF)), _full((NF, NF)), _full((NF, NF))],
        out_specs=[_rows(BQ), _rows(BQ), _rows(BKV), _STSPEC],
        out_shape=[jax.ShapeDtypeStruct((NT, NF), f32),
                   jax.ShapeDtypeStruct((NT, NF), jnp.bfloat16),
                   jax.ShapeDtypeStruct((NKV, NF), f32), _ST],
    )(flat, flat_even, W_p1, W_kv, W_q)

    k, v = pl.pallas_call(
        _s2,
        grid=(NKV // BQ,),
        in_specs=[_rows(BQ), _STSPEC, _full((1, NF)), _full((1, NF)),
                  _full((NF, NF)), _full((NF, NF))],
        out_specs=[_rows(BQ), _rows(BQ)],
        out_shape=[jax.ShapeDtypeStruct((NKV, NF), jnp.bfloat16),
                   jax.ShapeDtypeStruct((NKV, NF), jnp.bfloat16)],
    )(kvp, st_kv, g_kv2, b_kv2, W_k, W_v)

    y, st_y = pl.pallas_call(
        _s3,
        grid=(NBQ,),
        in_specs=[_rows(BQ), _full((NKV, NF)), _full((NKV, NF)),
                  _full((NF, NF))],
        out_specs=[_rows(BQ), _STSPEC],
        out_shape=[jax.ShapeDtypeStruct((NT, NF), f32), _ST],
    )(q, k, v, W_trans)

    x2, st_x2 = pl.pallas_call(
        _s4,
        grid=(NBQ,),
        in_specs=[_rows(BQ), _rows(BQ), _STSPEC, _full((1, NF)),
                  _full((1, NF))],
        out_specs=[_rows(BQ), _STSPEC],
        out_shape=[jax.ShapeDtypeStruct((NT, NF), f32), _ST],
    )(x, y, st_y, g_an2, b_an2)

    h1, st_h1 = pl.pallas_call(
        _s5,
        grid=(NBQ,),
        in_specs=[_rows(BQ), _STSPEC, _full((1, NF)), _full((1, NF)),
                  _full((NF, NF))],
        out_specs=[_rows(BQ), _STSPEC],
        out_shape=[jax.ShapeDtypeStruct((NT, NF), f32), _ST],
    )(x2, st_x2, g12, b12, W_r1)

    x3, st_x3 = pl.pallas_call(
        _s6,
        grid=(NBQ,),
        in_specs=[_rows(BQ), _rows(BQ), _STSPEC, _full((1, NF)),
                  _full((1, NF)), _full((NF, NF))],
        out_specs=[_rows(BQ), _STSPEC],
        out_shape=[jax.ShapeDtypeStruct((NT, NF), f32), _ST],
    )(h1, x2, st_h1, g22, b22, W_r2)

    out = pl.pallas_call(
        _s7,
        grid=(NBQ,),
        in_specs=[_rows(BQ), _STSPEC, _full((1, NF)), _full((1, NF))],
        out_specs=_rows(BQ),
        out_shape=jax.ShapeDtypeStruct((NT, NF), f32),
    )(x3, st_x3, g32, b32)

    return out


# speedup vs baseline: 1.3666x; 1.3487x over previous
import jax
import jax.numpy as jnp
from jax import lax
from jax.experimental import pallas as pl
from jax.experimental.pallas import tpu as pltpu

NT = 8192
NKV = NT // 2
NF_IN = 128
NF = 256
EPS = 1e-4

BL = 512
NBL = NT // BL
BKV = NKV // NBL
BA = 256
NBA = NT // BA


def _bf(a):
    return a.astype(jnp.bfloat16)


def _dot(a, b):
    return jnp.dot(_bf(a), _bf(b), preferred_element_type=jnp.float32)


def _affine(st, n, g, b):
    s0, s1 = st
    mu = s0 / n
    var = s1 / n - mu * mu
    a = g * lax.rsqrt(var + EPS)
    c = b - mu * a
    return a, c


def _zst():
    return (jnp.zeros((1, NF), jnp.float32), jnp.zeros((1, NF), jnp.float32))


def _acc(st, yb):
    return (st[0] + jnp.sum(yb, axis=0, keepdims=True),
            st[1] + jnp.sum(yb * yb, axis=0, keepdims=True))


def _mega(flat, flat_even, wp1, wkv, wq, wk, wv, wtrans, wr1, wr2, gb,
          out, X, KVP, K, V, E, F):
    def gbrow(r):
        return gb[r:r + 1, :]

    def p1(i, st):
        r = pl.ds(i * BL, BL)
        xb = _dot(flat[r, :], wp1[...])
        X[r, :] = xb
        kb = _dot(_dot(flat_even[pl.ds(i * BKV, BKV), :], wp1[...]),
                  wkv[...])
        KVP[pl.ds(i * BKV, BKV), :] = kb
        return _acc(st, kb)

    st_kv = lax.fori_loop(0, NBL, p1, _zst())
    a_kv, c_kv = _affine(st_kv, NKV, gbrow(0), gbrow(1))

    def p2(i, st):
        r = pl.ds(i * BL, BL)
        kn = KVP[r, :] * a_kv + c_kv
        K[r, :] = _bf(_dot(kn, wk[...]))
        V[r, :] = _bf(_dot(kn, wv[...]))
        return st

    lax.fori_loop(0, NKV // BL, p2, 0)

    def p3(i, st):
        r = pl.ds(i * BA, BA)
        qb = _bf(_dot(X[r, :], wq[...]))
        s = lax.dot_general(qb, K[...], (((1,), (1,)), ((), ())),
                            preferred_element_type=jnp.float32)
        m = jnp.max(s, axis=1, keepdims=True)
        p = jnp.exp(s - m)
        d = jnp.sum(p, axis=1, keepdims=True)
        xr = jnp.dot(_bf(p), V[...], preferred_element_type=jnp.float32) / d
        yb = _dot(xr, wtrans[...])
        E[r, :] = yb
        return _acc(st, yb)

    st_y = lax.fori_loop(0, NBA, p3, _zst())
    a_an, c_an = _affine(st_y, NT, gbrow(2), gbrow(3))

    def p4(i, st):
        r = pl.ds(i * BL, BL)
        x2 = X[r, :] + E[r, :] * a_an + c_an
        F[r, :] = x2
        return _acc(st, x2)

    st_x2 = lax.fori_loop(0, NBL, p4, _zst())
    a1, c1 = _affine(st_x2, NT, gbrow(4), gbrow(5))

    def p5(i, st):
        r = pl.ds(i * BL, BL)
        h1 = _dot(jnp.maximum(F[r, :] * a1 + c1, 0.0), wr1[...])
        E[r, :] = h1
        return _acc(st, h1)

    st_h1 = lax.fori_loop(0, NBL, p5, _zst())
    a2, c2 = _affine(st_h1, NT, gbrow(6), gbrow(7))

    def p6(i, st):
        r = pl.ds(i * BL, BL)
        x3 = F[r, :] + _dot(jnp.maximum(E[r, :] * a2 + c2, 0.0), wr2[...])
        out[r, :] = x3
        return _acc(st, x3)

    st_x3 = lax.fori_loop(0, NBL, p6, _zst())
    a3, c3 = _affine(st_x3, NT, gbrow(8), gbrow(9))

    def p7(i, st):
        r = pl.ds(i * BL, BL)
        out[r, :] = jnp.maximum(out[r, :] * a3 + c3, 0.0)
        return st

    lax.fori_loop(0, NBL, p7, 0)


def kernel(flat, cu_seqlens, W_p1, W_kv, g_kv, b_kv, W_q, W_k, W_v,
           W_trans, g_an, b_an, g1, b1, W_r1, g2, b2, W_r2, g3, b3):
    del cu_seqlens
    f32 = jnp.float32
    bf16 = jnp.bfloat16
    flat_even = flat[::2]
    gb = jnp.stack([g_kv, b_kv, g_an, b_an, g1, b1, g2, b2, g3, b3])

    return pl.pallas_call(
        _mega,
        grid=(1,),
        out_shape=jax.ShapeDtypeStruct((NT, NF), f32),
        scratch_shapes=[
            pltpu.VMEM((NT, NF), f32),
            pltpu.VMEM((NKV, NF), f32),
            pltpu.VMEM((NKV, NF), bf16),
            pltpu.VMEM((NKV, NF), bf16),
            pltpu.VMEM((NT, NF), f32),
            pltpu.VMEM((NT, NF), f32),
        ],
    )(flat, flat_even, W_p1, W_kv, W_q, W_k, W_v, W_trans, W_r1, W_r2, gb)
